# initial kernel scaffold (unmeasured)
import jax
import jax.numpy as jnp
from jax import lax
from jax.experimental import pallas as pl
from jax.experimental.pallas import tpu as pltpu

M = 8192
D = 2048
BLOCK_M = 512
N_BLOCKS = M // BLOCK_M
EPS = 1e-6


def kernel(partial, resid, gamma):
    partial = partial.reshape(M, D)
    gamma = gamma.reshape(1, D)

    def body(partial_ref, resid_ref, gamma_ref, out_ref,
             recv_buf, send_sems, recv_sems):
        c = pl.program_id(0)
        my_x = lax.axis_index("x")
        my_y = lax.axis_index("y")
        my_z = lax.axis_index("z")
        peer = (my_x, my_y, 1 - my_z)

        slot = lax.rem(c, 2)
        rdma = pltpu.make_async_remote_copy(
            src_ref=partial_ref,
            dst_ref=recv_buf.at[slot],
            send_sem=send_sems.at[slot],
            recv_sem=recv_sems.at[slot],
            device_id=peer,
            device_id_type=pl.DeviceIdType.MESH,
        )
        rdma.start()
        t = partial_ref[...] + resid_ref[...]
        rdma.wait_recv()
        y = t + recv_buf[slot]
        rms = jnp.sqrt(jnp.mean(y * y, axis=-1, keepdims=True) + EPS)
        out_ref[...] = y / rms * gamma_ref[...]
        rdma.wait_send()

    return pl.pallas_call(
        body,
        grid=(N_BLOCKS,),
        in_specs=[
            pl.BlockSpec((BLOCK_M, D), lambda c: (c, 0)),
            pl.BlockSpec((BLOCK_M, D), lambda c: (c, 0)),
            pl.BlockSpec((1, D), lambda c: (0, 0)),
        ],
        out_specs=pl.BlockSpec((BLOCK_M, D), lambda c: (c, 0)),
        out_shape=jax.ShapeDtypeStruct((M, D), jnp.float32),
        scratch_shapes=[
            pltpu.VMEM((2, BLOCK_M, D), jnp.float32),
            pltpu.SemaphoreType.DMA((2,)),
            pltpu.SemaphoreType.DMA((2,)),
        ],
    )(partial, resid, gamma)


# baseline (device time: 818613 ns/iter reference)
import jax
import jax.numpy as jnp
from jax import lax
from jax.experimental import pallas as pl
from jax.experimental.pallas import tpu as pltpu

M = 8192
D = 2048
BLOCK_M = 512
N_BLOCKS = M // BLOCK_M
EPS = 1e-6


def kernel(partial, resid, gamma):
    partial = partial.reshape(M, D)
    gamma = gamma.reshape(1, D)

    def body(partial_ref, resid_ref, gamma_ref, out_ref,
             recv_buf, send_sems, recv_sems):
        c = pl.program_id(0)
        my_x = lax.axis_index("x")
        my_y = lax.axis_index("y")
        my_z = lax.axis_index("z")
        peer = (my_x, my_y, 1 - my_z)

        slot = lax.rem(c, 2)
        rdma = pltpu.make_async_remote_copy(
            src_ref=partial_ref,
            dst_ref=recv_buf.at[slot],
            send_sem=send_sems.at[slot],
            recv_sem=recv_sems.at[slot],
            device_id=peer,
            device_id_type=pl.DeviceIdType.MESH,
        )
        rdma.start()
        t = partial_ref[...] + resid_ref[...]
        rdma.wait_recv()
        y = t + recv_buf[slot]
        rms = jnp.sqrt(jnp.mean(y * y, axis=-1, keepdims=True) + EPS)
        out_ref[...] = y / rms * gamma_ref[...]
        rdma.wait_send()

    return pl.pallas_call(
        body,
        grid=(N_BLOCKS,),
        in_specs=[
            pl.BlockSpec((BLOCK_M, D), lambda c: (c, 0)),
            pl.BlockSpec((BLOCK_M, D), lambda c: (c, 0)),
            pl.BlockSpec((1, D), lambda c: (0, 0)),
        ],
        out_specs=pl.BlockSpec((BLOCK_M, D), lambda c: (c, 0)),
        out_shape=jax.ShapeDtypeStruct((M, D), jnp.float32),
        scratch_shapes=[
            pltpu.VMEM((2, BLOCK_M, D), jnp.float32),
            pltpu.SemaphoreType.DMA((2,)),
            pltpu.SemaphoreType.DMA((2,)),
        ],
        compiler_params=pltpu.CompilerParams(
            vmem_limit_bytes=100 * 1024 * 1024,
        ),
    )(partial, resid, gamma)


# device time: 818407 ns/iter; 1.0003x vs baseline; 1.0003x over previous
import jax
import jax.numpy as jnp
from jax import lax
from jax.experimental import pallas as pl
from jax.experimental.pallas import tpu as pltpu

M = 8192
D = 2048
BLOCK_M = 512
N_BLOCKS = M // BLOCK_M
EPS = 1e-6


def kernel(partial, resid, gamma):
    partial = partial.reshape(M, D)
    gamma = gamma.reshape(1, D)

    def body(partial_ref, resid_ref, gamma_ref, out_ref,
             recv_buf, send_sems, recv_sems):
        c = pl.program_id(0)
        my_x = lax.axis_index("x")
        my_y = lax.axis_index("y")
        my_z = lax.axis_index("z")
        peer = (my_x, my_y, 1 - my_z)

        slot = lax.rem(c, 2)
        HALF = BLOCK_M // 2
        rdmas = []
        for h in range(2):
            rdmas.append(pltpu.make_async_remote_copy(
                src_ref=partial_ref.at[pl.ds(h * HALF, HALF), :],
                dst_ref=recv_buf.at[slot, pl.ds(h * HALF, HALF), :],
                send_sem=send_sems.at[slot, h],
                recv_sem=recv_sems.at[slot, h],
                device_id=peer,
                device_id_type=pl.DeviceIdType.MESH,
            ))
        for r in rdmas:
            r.start()
        t = partial_ref[...] + resid_ref[...]
        for r in rdmas:
            r.wait_recv()
        y = t + recv_buf[slot]
        rms = jnp.sqrt(jnp.mean(y * y, axis=-1, keepdims=True) + EPS)
        out_ref[...] = y / rms * gamma_ref[...]
        for r in rdmas:
            r.wait_send()

    return pl.pallas_call(
        body,
        grid=(N_BLOCKS,),
        in_specs=[
            pl.BlockSpec((BLOCK_M, D), lambda c: (c, 0)),
            pl.BlockSpec((BLOCK_M, D), lambda c: (c, 0)),
            pl.BlockSpec((1, D), lambda c: (0, 0)),
        ],
        out_specs=pl.BlockSpec((BLOCK_M, D), lambda c: (c, 0)),
        out_shape=jax.ShapeDtypeStruct((M, D), jnp.float32),
        scratch_shapes=[
            pltpu.VMEM((2, BLOCK_M, D), jnp.float32),
            pltpu.SemaphoreType.DMA((2, 2)),
            pltpu.SemaphoreType.DMA((2, 2)),
        ],
        compiler_params=pltpu.CompilerParams(
            vmem_limit_bytes=100 * 1024 * 1024,
        ),
    )(partial, resid, gamma)


# device time: 817981 ns/iter; 1.0008x vs baseline; 1.0005x over previous
import jax
import jax.numpy as jnp
from jax import lax
from jax.experimental import pallas as pl
from jax.experimental.pallas import tpu as pltpu

M = 8192
D = 2048
BLOCK_M = 512
N_BLOCKS = M // BLOCK_M
EPS = 1e-6


def kernel(partial, resid, gamma):
    partial = partial.reshape(M, D)
    gamma = gamma.reshape(1, D)

    def body(partial_ref, resid_ref, gamma_ref, out_ref,
             recv_buf, send_sems, recv_sems):
        c = pl.program_id(0)
        my_x = lax.axis_index("x")
        my_y = lax.axis_index("y")
        my_z = lax.axis_index("z")
        peers = [
            (my_x, my_y, 1 - my_z),
            (1 - my_x, my_y, 1 - my_z),
        ]

        slot = lax.rem(c, 2)
        HALF = BLOCK_M // 2
        rdmas = []
        for h, peer in enumerate(peers):
            rdmas.append(pltpu.make_async_remote_copy(
                src_ref=partial_ref.at[pl.ds(h * HALF, HALF), :],
                dst_ref=recv_buf.at[slot, pl.ds(h * HALF, HALF), :],
                send_sem=send_sems.at[slot, h],
                recv_sem=recv_sems.at[slot, h],
                device_id=peer,
                device_id_type=pl.DeviceIdType.MESH,
            ))
        for r in rdmas:
            r.start()
        t = partial_ref[...] + resid_ref[...]
        for r in rdmas:
            r.wait_recv()
        y = t + recv_buf[slot]
        rms = jnp.sqrt(jnp.mean(y * y, axis=-1, keepdims=True) + EPS)
        out_ref[...] = y / rms * gamma_ref[...]
        for r in rdmas:
            r.wait_send()

    return pl.pallas_call(
        body,
        grid=(N_BLOCKS,),
        in_specs=[
            pl.BlockSpec((BLOCK_M, D), lambda c: (c, 0)),
            pl.BlockSpec((BLOCK_M, D), lambda c: (c, 0)),
            pl.BlockSpec((1, D), lambda c: (0, 0)),
        ],
        out_specs=pl.BlockSpec((BLOCK_M, D), lambda c: (c, 0)),
        out_shape=jax.ShapeDtypeStruct((M, D), jnp.float32),
        scratch_shapes=[
            pltpu.VMEM((2, BLOCK_M, D), jnp.float32),
            pltpu.SemaphoreType.DMA((2, 2)),
            pltpu.SemaphoreType.DMA((2, 2)),
        ],
        compiler_params=pltpu.CompilerParams(
            vmem_limit_bytes=100 * 1024 * 1024,
        ),
    )(partial, resid, gamma)


# device time: 389352 ns/iter; 2.1025x vs baseline; 2.1009x over previous
import jax
import jax.numpy as jnp
from jax import lax
from jax.experimental import pallas as pl
from jax.experimental.pallas import tpu as pltpu

M = 8192
D = 2048
BLOCK_M = 512
N_BLOCKS = M // BLOCK_M
QUARTER = BLOCK_M // 4
HALF_Q = QUARTER // 2
N_SLOTS = 4
EPS = 1e-6

SEM_Z = 0
SEM_FULL_L = 1
SEM_FULL_R = 2
SEM_HALF_L = 3
SEM_HALF_R = 4
N_FLOWS = 5


def kernel(partial, resid, gamma):
    partial = partial.reshape(M, D)
    gamma = gamma.reshape(1, D)

    def body(partial_ref, resid_ref, gamma_ref, out_ref,
             asm, t_buf, send_sems, recv_sems):
        c = pl.program_id(0)
        my_x = lax.axis_index("x")
        my_y = lax.axis_index("y")
        my_z = lax.axis_index("z")

        p = my_x + 3 * my_y - 2 * my_x * my_y

        def ring_coords(q):
            qh = q // 2
            ql = lax.rem(q, 4) % 2
            return (qh + ql - 2 * qh * ql, qh)

        pr = lax.rem(p + 1, 4)
        plft = lax.rem(p + 3, 4)
        rx, ry = ring_coords(pr)
        lx, ly = ring_coords(plft)
        right_dev = (rx, ry, my_z)
        left_dev = (lx, ly, my_z)
        zpeer_dev = (my_x, my_y, 1 - my_z)

        s0 = lax.rem(c, N_SLOTS)
        s1 = lax.rem(c + N_SLOTS - 1, N_SLOTS)
        s2 = lax.rem(c + N_SLOTS - 2, N_SLOTS)

        def quarter_rows(ref_slot, q, off, rows):
            return ref_slot.at[pl.ds(q * QUARTER + off, rows), :]

        def copy(src, dst, ssem, dev, rsem_idx, slot):
            return pltpu.make_async_remote_copy(
                src_ref=src,
                dst_ref=dst,
                send_sem=send_sems.at[ssem],
                recv_sem=recv_sems.at[slot, rsem_idx],
                device_id=dev,
                device_id_type=pl.DeviceIdType.MESH,
            )

        @pl.when(jnp.logical_and(c >= 1, c <= N_BLOCKS))
        def _():
            copy(quarter_rows(asm.at[s1], p, 0, QUARTER),
                 quarter_rows(asm.at[s1], p, 0, QUARTER),
                 0, zpeer_dev, SEM_Z, s1).wait_recv()

        @pl.when(jnp.logical_and(c >= 2, c <= N_BLOCKS + 1))
        def _():
            copy(quarter_rows(asm.at[s2], plft, 0, QUARTER),
                 quarter_rows(asm.at[s2], plft, 0, QUARTER),
                 0, left_dev, SEM_FULL_L, s2).wait_recv()
            copy(quarter_rows(asm.at[s2], pr, 0, QUARTER),
                 quarter_rows(asm.at[s2], pr, 0, QUARTER),
                 0, right_dev, SEM_FULL_R, s2).wait_recv()

        sends = []
        @pl.when(c < N_BLOCKS)
        def _():
            r = copy(quarter_rows(partial_ref, p, 0, QUARTER),
                     quarter_rows(asm.at[s0], p, 0, QUARTER),
                     0, zpeer_dev, SEM_Z, s0)
            r.start()

        @pl.when(jnp.logical_and(c >= 1, c <= N_BLOCKS))
        def _():
            r1 = copy(quarter_rows(asm.at[s1], p, 0, QUARTER),
                      quarter_rows(asm.at[s1], p, 0, QUARTER),
                      1, right_dev, SEM_FULL_L, s1)
            r1.start()
            r2 = copy(quarter_rows(asm.at[s1], p, 0, QUARTER),
                      quarter_rows(asm.at[s1], p, 0, QUARTER),
                      2, left_dev, SEM_FULL_R, s1)
            r2.start()

        @pl.when(jnp.logical_and(c >= 2, c <= N_BLOCKS + 1))
        def _():
            r3 = copy(quarter_rows(asm.at[s2], plft, 0, HALF_Q),
                      quarter_rows(asm.at[s2], plft, 0, HALF_Q),
                      3, right_dev, SEM_HALF_L, s2)
            r3.start()
            r4 = copy(quarter_rows(asm.at[s2], pr, HALF_Q, HALF_Q),
                      quarter_rows(asm.at[s2], pr, HALF_Q, HALF_Q),
                      4, left_dev, SEM_HALF_R, s2)
            r4.start()

        @pl.when(c < N_BLOCKS)
        def _():
            t_buf[s0] = partial_ref[...] + resid_ref[...]

        @pl.when(jnp.logical_and(c >= 2, c <= N_BLOCKS + 1))
        def _():
            popp = lax.rem(p + 2, 4)
            copy(quarter_rows(asm.at[s2], popp, 0, HALF_Q),
                 quarter_rows(asm.at[s2], popp, 0, HALF_Q),
                 0, left_dev, SEM_HALF_L, s2).wait_recv()
            copy(quarter_rows(asm.at[s2], popp, HALF_Q, HALF_Q),
                 quarter_rows(asm.at[s2], popp, HALF_Q, HALF_Q),
                 0, right_dev, SEM_HALF_R, s2).wait_recv()
            y = t_buf[s2] + asm[s2]
            rms = jnp.sqrt(jnp.mean(y * y, axis=-1, keepdims=True) + EPS)
            out_ref[...] = y / rms * gamma_ref[...]

        @pl.when(c < N_BLOCKS)
        def _():
            copy(quarter_rows(partial_ref, p, 0, QUARTER),
                 quarter_rows(asm.at[s0], p, 0, QUARTER),
                 0, zpeer_dev, SEM_Z, s0).wait_send()

        @pl.when(jnp.logical_and(c >= 1, c <= N_BLOCKS))
        def _():
            copy(quarter_rows(asm.at[s1], p, 0, QUARTER),
                 quarter_rows(asm.at[s1], p, 0, QUARTER),
                 1, right_dev, SEM_FULL_L, s1).wait_send()
            copy(quarter_rows(asm.at[s1], p, 0, QUARTER),
                 quarter_rows(asm.at[s1], p, 0, QUARTER),
                 2, left_dev, SEM_FULL_R, s1).wait_send()

        @pl.when(jnp.logical_and(c >= 2, c <= N_BLOCKS + 1))
        def _():
            copy(quarter_rows(asm.at[s2], plft, 0, HALF_Q),
                 quarter_rows(asm.at[s2], plft, 0, HALF_Q),
                 3, right_dev, SEM_HALF_L, s2).wait_send()
            copy(quarter_rows(asm.at[s2], pr, HALF_Q, HALF_Q),
                 quarter_rows(asm.at[s2], pr, HALF_Q, HALF_Q),
                 4, left_dev, SEM_HALF_R, s2).wait_send()

    def in_idx(c):
        return (jnp.minimum(c, N_BLOCKS - 1), 0)

    def out_idx(c):
        return (jnp.clip(c - 2, 0, N_BLOCKS - 1), 0)

    return pl.pallas_call(
        body,
        grid=(N_BLOCKS + 2,),
        in_specs=[
            pl.BlockSpec((BLOCK_M, D), in_idx),
            pl.BlockSpec((BLOCK_M, D), in_idx),
            pl.BlockSpec((1, D), lambda c: (0, 0)),
        ],
        out_specs=pl.BlockSpec((BLOCK_M, D), out_idx),
        out_shape=jax.ShapeDtypeStruct((M, D), jnp.float32),
        scratch_shapes=[
            pltpu.VMEM((N_SLOTS, BLOCK_M, D), jnp.float32),
            pltpu.VMEM((N_SLOTS, BLOCK_M, D), jnp.float32),
            pltpu.SemaphoreType.DMA((N_FLOWS,)),
            pltpu.SemaphoreType.DMA((N_SLOTS, N_FLOWS)),
        ],
        compiler_params=pltpu.CompilerParams(
            vmem_limit_bytes=100 * 1024 * 1024,
        ),
    )(partial, resid, gamma)


# device time: 355118 ns/iter; 2.3052x vs baseline; 1.0964x over previous
import jax
import jax.numpy as jnp
from jax import lax
from jax.experimental import pallas as pl
from jax.experimental.pallas import tpu as pltpu

M = 8192
D = 2048
BLOCK_M = 512
N_BLOCKS = M // BLOCK_M
QUARTER = BLOCK_M // 4
HALF_Q = QUARTER // 2
N_SLOTS = 5
EPS = 1e-6

SEM_Z = 0
SEM_FULL_L = 1
SEM_FULL_R = 2
SEM_HALF_L = 3
SEM_HALF_R = 4
N_FLOWS = 5


def kernel(partial, resid, gamma):
    partial = partial.reshape(M, D)
    gamma = gamma.reshape(1, D)

    def body(partial_ref, resid_ref, gamma_ref, partial_cmp_ref, out_ref,
             asm, send_sems, recv_sems):
        c = pl.program_id(0)
        my_x = lax.axis_index("x")
        my_y = lax.axis_index("y")
        my_z = lax.axis_index("z")

        p = my_x + 3 * my_y - 2 * my_x * my_y

        def ring_coords(q):
            qh = q // 2
            ql = lax.rem(q, 4) % 2
            return (qh + ql - 2 * qh * ql, qh)

        pr = lax.rem(p + 1, 4)
        plft = lax.rem(p + 3, 4)
        rx, ry = ring_coords(pr)
        lx, ly = ring_coords(plft)
        right_dev = (rx, ry, my_z)
        left_dev = (lx, ly, my_z)
        zpeer_dev = (my_x, my_y, 1 - my_z)

        s0 = lax.rem(c, N_SLOTS)
        s1 = lax.rem(c + N_SLOTS - 1, N_SLOTS)
        s2 = lax.rem(c + N_SLOTS - 2, N_SLOTS)
        s3 = lax.rem(c + N_SLOTS - 3, N_SLOTS)

        def quarter_rows(ref_slot, q, off, rows):
            return ref_slot.at[pl.ds(q * QUARTER + off, rows), :]

        def copy(src, dst, ssem, dev, rsem_idx, slot):
            return pltpu.make_async_remote_copy(
                src_ref=src,
                dst_ref=dst,
                send_sem=send_sems.at[ssem],
                recv_sem=recv_sems.at[slot, rsem_idx],
                device_id=dev,
                device_id_type=pl.DeviceIdType.MESH,
            )

        @pl.when(jnp.logical_and(c >= 1, c <= N_BLOCKS))
        def _():
            copy(quarter_rows(asm.at[s1], p, 0, QUARTER),
                 quarter_rows(asm.at[s1], p, 0, QUARTER),
                 0, zpeer_dev, SEM_Z, s1).wait_recv()

        @pl.when(jnp.logical_and(c >= 2, c <= N_BLOCKS + 1))
        def _():
            copy(quarter_rows(asm.at[s2], plft, 0, QUARTER),
                 quarter_rows(asm.at[s2], plft, 0, QUARTER),
                 0, left_dev, SEM_FULL_L, s2).wait_recv()
            copy(quarter_rows(asm.at[s2], pr, 0, QUARTER),
                 quarter_rows(asm.at[s2], pr, 0, QUARTER),
                 0, right_dev, SEM_FULL_R, s2).wait_recv()

        sends = []
        @pl.when(c < N_BLOCKS)
        def _():
            r = copy(quarter_rows(partial_ref, p, 0, QUARTER),
                     quarter_rows(asm.at[s0], p, 0, QUARTER),
                     0, zpeer_dev, SEM_Z, s0)
            r.start()

        @pl.when(jnp.logical_and(c >= 1, c <= N_BLOCKS))
        def _():
            r1 = copy(quarter_rows(asm.at[s1], p, 0, QUARTER),
                      quarter_rows(asm.at[s1], p, 0, QUARTER),
                      1, right_dev, SEM_FULL_L, s1)
            r1.start()
            r2 = copy(quarter_rows(asm.at[s1], p, 0, QUARTER),
                      quarter_rows(asm.at[s1], p, 0, QUARTER),
                      2, left_dev, SEM_FULL_R, s1)
            r2.start()

        @pl.when(jnp.logical_and(c >= 2, c <= N_BLOCKS + 1))
        def _():
            r3 = copy(quarter_rows(asm.at[s2], plft, 0, HALF_Q),
                      quarter_rows(asm.at[s2], plft, 0, HALF_Q),
                      3, right_dev, SEM_HALF_L, s2)
            r3.start()
            r4 = copy(quarter_rows(asm.at[s2], pr, HALF_Q, HALF_Q),
                      quarter_rows(asm.at[s2], pr, HALF_Q, HALF_Q),
                      4, left_dev, SEM_HALF_R, s2)
            r4.start()

        @pl.when(jnp.logical_and(c >= 3, c <= N_BLOCKS + 2))
        def _():
            popp = lax.rem(p + 2, 4)
            copy(quarter_rows(asm.at[s3], popp, 0, HALF_Q),
                 quarter_rows(asm.at[s3], popp, 0, HALF_Q),
                 0, left_dev, SEM_HALF_L, s3).wait_recv()
            copy(quarter_rows(asm.at[s3], popp, HALF_Q, HALF_Q),
                 quarter_rows(asm.at[s3], popp, HALF_Q, HALF_Q),
                 0, right_dev, SEM_HALF_R, s3).wait_recv()
            y = partial_cmp_ref[...] + resid_ref[...] + asm[s3]
            rms = jnp.sqrt(jnp.mean(y * y, axis=-1, keepdims=True) + EPS)
            out_ref[...] = y / rms * gamma_ref[...]

        @pl.when(c < N_BLOCKS)
        def _():
            copy(quarter_rows(partial_ref, p, 0, QUARTER),
                 quarter_rows(asm.at[s0], p, 0, QUARTER),
                 0, zpeer_dev, SEM_Z, s0).wait_send()

        @pl.when(jnp.logical_and(c >= 1, c <= N_BLOCKS))
        def _():
            copy(quarter_rows(asm.at[s1], p, 0, QUARTER),
                 quarter_rows(asm.at[s1], p, 0, QUARTER),
                 1, right_dev, SEM_FULL_L, s1).wait_send()
            copy(quarter_rows(asm.at[s1], p, 0, QUARTER),
                 quarter_rows(asm.at[s1], p, 0, QUARTER),
                 2, left_dev, SEM_FULL_R, s1).wait_send()

        @pl.when(jnp.logical_and(c >= 2, c <= N_BLOCKS + 1))
        def _():
            copy(quarter_rows(asm.at[s2], plft, 0, HALF_Q),
                 quarter_rows(asm.at[s2], plft, 0, HALF_Q),
                 3, right_dev, SEM_HALF_L, s2).wait_send()
            copy(quarter_rows(asm.at[s2], pr, HALF_Q, HALF_Q),
                 quarter_rows(asm.at[s2], pr, HALF_Q, HALF_Q),
                 4, left_dev, SEM_HALF_R, s2).wait_send()

    def in_idx(c):
        return (jnp.minimum(c, N_BLOCKS - 1), 0)

    def out_idx(c):
        return (jnp.clip(c - 3, 0, N_BLOCKS - 1), 0)

    return pl.pallas_call(
        body,
        grid=(N_BLOCKS + 3,),
        in_specs=[
            pl.BlockSpec((BLOCK_M, D), in_idx),
            pl.BlockSpec((BLOCK_M, D), out_idx),
            pl.BlockSpec((1, D), lambda c: (0, 0)),
            pl.BlockSpec((BLOCK_M, D), out_idx),
        ],
        out_specs=pl.BlockSpec((BLOCK_M, D), out_idx),
        out_shape=jax.ShapeDtypeStruct((M, D), jnp.float32),
        scratch_shapes=[
            pltpu.VMEM((N_SLOTS, BLOCK_M, D), jnp.float32),
            pltpu.SemaphoreType.DMA((N_FLOWS,)),
            pltpu.SemaphoreType.DMA((N_SLOTS, N_FLOWS)),
        ],
        compiler_params=pltpu.CompilerParams(
            vmem_limit_bytes=100 * 1024 * 1024,
        ),
    )(partial, resid, gamma, partial)


# device time: 336268 ns/iter; 2.4344x vs baseline; 1.0561x over previous
import jax
import jax.numpy as jnp
from jax import lax
from jax.experimental import pallas as pl
from jax.experimental.pallas import tpu as pltpu

M = 8192
D = 2048
BLOCK_M = 512
N_BLOCKS = M // BLOCK_M
QUARTER = BLOCK_M // 4
HALF_Q = QUARTER // 2
N_SLOTS = 5
EPS = 1e-6

SEM_Z = 0
SEM_FULL_L = 1
SEM_FULL_R = 2
SEM_HALF_L = 3
SEM_HALF_R = 4
SEM_Z2 = 5
N_FLOWS = 6
Z_EXTRA = 40
RING_A = 48


def kernel(partial, resid, gamma):
    partial = partial.reshape(M, D)
    gamma = gamma.reshape(1, D)

    def body(partial_ref, resid_ref, gamma_ref, partial_cmp_ref, out_ref,
             asm, send_sems, recv_sems):
        c = pl.program_id(0)
        my_x = lax.axis_index("x")
        my_y = lax.axis_index("y")
        my_z = lax.axis_index("z")

        p = my_x + 3 * my_y - 2 * my_x * my_y

        def ring_coords(q):
            qh = q // 2
            ql = lax.rem(q, 4) % 2
            return (qh + ql - 2 * qh * ql, qh)

        pr = lax.rem(p + 1, 4)
        plft = lax.rem(p + 3, 4)
        rx, ry = ring_coords(pr)
        lx, ly = ring_coords(plft)
        right_dev = (rx, ry, my_z)
        left_dev = (lx, ly, my_z)
        zpeer_dev = (my_x, my_y, 1 - my_z)

        s0 = lax.rem(c, N_SLOTS)
        s1 = lax.rem(c + N_SLOTS - 1, N_SLOTS)
        s2 = lax.rem(c + N_SLOTS - 2, N_SLOTS)
        s3 = lax.rem(c + N_SLOTS - 3, N_SLOTS)

        def quarter_rows(ref_slot, q, off, rows):
            return ref_slot.at[pl.ds(q * QUARTER + off, rows), :]

        def copy(src, dst, ssem, dev, rsem_idx, slot):
            return pltpu.make_async_remote_copy(
                src_ref=src,
                dst_ref=dst,
                send_sem=send_sems.at[ssem],
                recv_sem=recv_sems.at[slot, rsem_idx],
                device_id=dev,
                device_id_type=pl.DeviceIdType.MESH,
            )

        @pl.when(jnp.logical_and(c >= 1, c <= N_BLOCKS))
        def _():
            copy(quarter_rows(asm.at[s1], p, 0, QUARTER),
                 quarter_rows(asm.at[s1], p, 0, QUARTER),
                 0, zpeer_dev, SEM_Z, s1).wait_recv()

        @pl.when(jnp.logical_and(c >= 2, c <= N_BLOCKS + 1))
        def _():
            copy(quarter_rows(asm.at[s2], plft, 0, QUARTER),
                 quarter_rows(asm.at[s2], plft, 0, QUARTER),
                 0, left_dev, SEM_FULL_L, s2).wait_recv()
            copy(quarter_rows(asm.at[s2], pr, 0, QUARTER),
                 quarter_rows(asm.at[s2], pr, 0, QUARTER),
                 0, right_dev, SEM_FULL_R, s2).wait_recv()

        popp = lax.rem(p + 2, 4)

        @pl.when(c < N_BLOCKS)
        def _():
            r = copy(quarter_rows(partial_ref, p, 0, QUARTER),
                     quarter_rows(asm.at[s0], p, 0, QUARTER),
                     0, zpeer_dev, SEM_Z, s0)
            r.start()
            r2 = copy(quarter_rows(partial_ref, popp, 0, Z_EXTRA),
                      quarter_rows(asm.at[s0], popp, 0, Z_EXTRA),
                      5, zpeer_dev, SEM_Z2, s0)
            r2.start()

        @pl.when(jnp.logical_and(c >= 1, c <= N_BLOCKS))
        def _():
            r1 = copy(quarter_rows(asm.at[s1], p, 0, QUARTER),
                      quarter_rows(asm.at[s1], p, 0, QUARTER),
                      1, right_dev, SEM_FULL_L, s1)
            r1.start()
            r2 = copy(quarter_rows(asm.at[s1], p, 0, QUARTER),
                      quarter_rows(asm.at[s1], p, 0, QUARTER),
                      2, left_dev, SEM_FULL_R, s1)
            r2.start()

        @pl.when(jnp.logical_and(c >= 2, c <= N_BLOCKS + 1))
        def _():
            r3 = copy(quarter_rows(asm.at[s2], plft, Z_EXTRA, RING_A),
                      quarter_rows(asm.at[s2], plft, Z_EXTRA, RING_A),
                      3, right_dev, SEM_HALF_L, s2)
            r3.start()
            r4 = copy(quarter_rows(asm.at[s2], pr, Z_EXTRA + RING_A,
                                   QUARTER - Z_EXTRA - RING_A),
                      quarter_rows(asm.at[s2], pr, Z_EXTRA + RING_A,
                                   QUARTER - Z_EXTRA - RING_A),
                      4, left_dev, SEM_HALF_R, s2)
            r4.start()

        @pl.when(jnp.logical_and(c >= 3, c <= N_BLOCKS + 2))
        def _():
            copy(quarter_rows(asm.at[s3], popp, 0, Z_EXTRA),
                 quarter_rows(asm.at[s3], popp, 0, Z_EXTRA),
                 0, zpeer_dev, SEM_Z2, s3).wait_recv()
            copy(quarter_rows(asm.at[s3], popp, Z_EXTRA, RING_A),
                 quarter_rows(asm.at[s3], popp, Z_EXTRA, RING_A),
                 0, left_dev, SEM_HALF_L, s3).wait_recv()
            copy(quarter_rows(asm.at[s3], popp, Z_EXTRA + RING_A,
                              QUARTER - Z_EXTRA - RING_A),
                 quarter_rows(asm.at[s3], popp, Z_EXTRA + RING_A,
                              QUARTER - Z_EXTRA - RING_A),
                 0, right_dev, SEM_HALF_R, s3).wait_recv()
            y = partial_cmp_ref[...] + resid_ref[...] + asm[s3]
            rms = jnp.sqrt(jnp.mean(y * y, axis=-1, keepdims=True) + EPS)
            out_ref[...] = y / rms * gamma_ref[...]

        @pl.when(c < N_BLOCKS)
        def _():
            copy(quarter_rows(partial_ref, p, 0, QUARTER),
                 quarter_rows(asm.at[s0], p, 0, QUARTER),
                 0, zpeer_dev, SEM_Z, s0).wait_send()
            copy(quarter_rows(partial_ref, popp, 0, Z_EXTRA),
                 quarter_rows(asm.at[s0], popp, 0, Z_EXTRA),
                 5, zpeer_dev, SEM_Z2, s0).wait_send()

        @pl.when(jnp.logical_and(c >= 1, c <= N_BLOCKS))
        def _():
            copy(quarter_rows(asm.at[s1], p, 0, QUARTER),
                 quarter_rows(asm.at[s1], p, 0, QUARTER),
                 1, right_dev, SEM_FULL_L, s1).wait_send()
            copy(quarter_rows(asm.at[s1], p, 0, QUARTER),
                 quarter_rows(asm.at[s1], p, 0, QUARTER),
                 2, left_dev, SEM_FULL_R, s1).wait_send()

        @pl.when(jnp.logical_and(c >= 2, c <= N_BLOCKS + 1))
        def _():
            copy(quarter_rows(asm.at[s2], plft, Z_EXTRA, RING_A),
                 quarter_rows(asm.at[s2], plft, Z_EXTRA, RING_A),
                 3, right_dev, SEM_HALF_L, s2).wait_send()
            copy(quarter_rows(asm.at[s2], pr, Z_EXTRA + RING_A,
                              QUARTER - Z_EXTRA - RING_A),
                 quarter_rows(asm.at[s2], pr, Z_EXTRA + RING_A,
                              QUARTER - Z_EXTRA - RING_A),
                 4, left_dev, SEM_HALF_R, s2).wait_send()

    def in_idx(c):
        return (jnp.minimum(c, N_BLOCKS - 1), 0)

    def out_idx(c):
        return (jnp.clip(c - 3, 0, N_BLOCKS - 1), 0)

    return pl.pallas_call(
        body,
        grid=(N_BLOCKS + 3,),
        in_specs=[
            pl.BlockSpec((BLOCK_M, D), in_idx),
            pl.BlockSpec((BLOCK_M, D), out_idx),
            pl.BlockSpec((1, D), lambda c: (0, 0)),
            pl.BlockSpec((BLOCK_M, D), out_idx),
        ],
        out_specs=pl.BlockSpec((BLOCK_M, D), out_idx),
        out_shape=jax.ShapeDtypeStruct((M, D), jnp.float32),
        scratch_shapes=[
            pltpu.VMEM((N_SLOTS, BLOCK_M, D), jnp.float32),
            pltpu.SemaphoreType.DMA((N_FLOWS,)),
            pltpu.SemaphoreType.DMA((N_SLOTS, N_FLOWS)),
        ],
        compiler_params=pltpu.CompilerParams(
            vmem_limit_bytes=100 * 1024 * 1024,
        ),
    )(partial, resid, gamma, partial)
